# baseline (speedup 1.0000x reference)
import functools

import jax
import jax.numpy as jnp
from jax import lax
from jax.experimental import pallas as pl
from jax.experimental.pallas import tpu as pltpu
from jax.experimental.pallas import tpu_sc as plsc

BATCH = 16384
DIM = 32
TLANE = 128
NU = 1000000
NV = 100000
NCOL_U = 782
NVA = 99968
NCOL_V = 781
NUNITS = NCOL_U + NCOL_V
PU = NCOL_U * 32
PV = NV // 4
PACK = 4
PDIM = 128
L = 16
NC, NS = 2, 16
NW = NC * NS
UPW = 49
BPW = BATCH // NW
CHUNK = 128
NCHUNK = BPW // CHUNK
GROUPS = CHUNK // L

_mesh = plsc.VectorSubcoreMesh(core_axis_name="c", subcore_axis_name="s")


@functools.partial(
    pl.kernel,
    mesh=_mesh,
    out_type=(
        jax.ShapeDtypeStruct((PU, PDIM), jnp.float32),
        jax.ShapeDtypeStruct((PV, PDIM), jnp.float32),
    ),
    compiler_params=pltpu.CompilerParams(needs_layout_passes=False),
    scratch_types=[
        pltpu.VMEM((2, DIM, TLANE), jnp.float32),
        pltpu.VMEM((2, DIM, TLANE), jnp.float32),
        pltpu.VMEM((8, PDIM), jnp.float32),
        pltpu.SemaphoreType.DMA,
        pltpu.SemaphoreType.DMA,
        pltpu.SemaphoreType.DMA,
        pltpu.SemaphoreType.DMA,
    ],
)
def _repack(ut_hbm, vt_hbm, vtail_hbm, pu_hbm, pv_hbm,
            colbuf, packbuf, tail_v, isem0, isem1, osem0, osem1):
    wid = lax.axis_index("s") * NC + lax.axis_index("c")
    isems = (isem0, isem1)
    osems = (osem0, osem1)
    lane = lax.iota(jnp.int32, L)

    def unit_of(t):
        return wid + NW * t

    def fire_in(t):
        u = unit_of(t)
        par = t & 1

        @pl.when((u < NCOL_U) & (t < UPW))
        def _():
            s0 = isems[0]
            s1 = isems[1]

            @pl.when(par == 0)
            def _():
                pltpu.async_copy(
                    ut_hbm.at[:, pl.ds(u * TLANE, TLANE)], colbuf.at[0], s0)

            @pl.when(par == 1)
            def _():
                pltpu.async_copy(
                    ut_hbm.at[:, pl.ds(u * TLANE, TLANE)], colbuf.at[1], s1)

        @pl.when((u >= NCOL_U) & (u < NUNITS) & (t < UPW))
        def _():
            c = u - NCOL_U

            @pl.when(par == 0)
            def _():
                pltpu.async_copy(
                    vt_hbm.at[:, pl.ds(c * TLANE, TLANE)], colbuf.at[0],
                    isems[0])

            @pl.when(par == 1)
            def _():
                pltpu.async_copy(
                    vt_hbm.at[:, pl.ds(c * TLANE, TLANE)], colbuf.at[1],
                    isems[1])

    def drain_in(t):
        u = unit_of(t)
        par = t & 1

        @pl.when((u < NUNITS) & (t < UPW))
        def _():
            @pl.when(par == 0)
            def _():
                pltpu.make_async_copy(
                    ut_hbm.at[:, pl.ds(0, TLANE)], colbuf.at[0], isems[0]
                ).wait()

            @pl.when(par == 1)
            def _():
                pltpu.make_async_copy(
                    ut_hbm.at[:, pl.ds(0, TLANE)], colbuf.at[1], isems[1]
                ).wait()

    def drain_out(t):
        u = unit_of(t)
        par = t & 1

        @pl.when((t >= 0) & (u < NUNITS))
        def _():
            @pl.when(par == 0)
            def _():
                pltpu.make_async_copy(
                    ut_hbm.at[:, pl.ds(0, TLANE)], packbuf.at[0], osems[0]
                ).wait()

            @pl.when(par == 1)
            def _():
                pltpu.make_async_copy(
                    ut_hbm.at[:, pl.ds(0, TLANE)], packbuf.at[1], osems[1]
                ).wait()

    d_even = lane
    d_odd = lane + 16

    def transpose(t):
        par = t & 1
        cb = colbuf.at[par]
        pb = packbuf.at[par]

        def qbody(q, carry):
            for j in range(8):
                dvec = d_even if j % 2 == 0 else d_odd
                r = jnp.full((L,), 0, jnp.int32) + (4 * q + j // 2)
                val = plsc.load_gather(cb, [dvec, r])
                pb[q, pl.ds(j * L, L)] = val
            return carry

        lax.fori_loop(0, DIM, qbody, 0)

    def fire_out(t):
        u = unit_of(t)
        par = t & 1

        @pl.when((u < NCOL_U) & (t < UPW))
        def _():
            @pl.when(par == 0)
            def _():
                pltpu.async_copy(
                    packbuf.at[0], pu_hbm.at[pl.ds(u * 32, 32)], osems[0])

            @pl.when(par == 1)
            def _():
                pltpu.async_copy(
                    packbuf.at[1], pu_hbm.at[pl.ds(u * 32, 32)], osems[1])

        @pl.when((u >= NCOL_U) & (u < NUNITS) & (t < UPW))
        def _():
            c = u - NCOL_U

            @pl.when(par == 0)
            def _():
                pltpu.async_copy(
                    packbuf.at[0], pv_hbm.at[pl.ds(c * 32, 32)], osems[0])

            @pl.when(par == 1)
            def _():
                pltpu.async_copy(
                    packbuf.at[1], pv_hbm.at[pl.ds(c * 32, 32)], osems[1])

    fire_in(0)

    def body(t, carry):
        fire_in(t + 1)
        drain_in(t)

        @pl.when(t >= 2)
        def _():
            drain_out(t - 2)

        transpose(t)
        fire_out(t)
        return carry

    lax.fori_loop(0, UPW, body, 0)
    drain_out(UPW - 2)
    drain_out(UPW - 1)

    @pl.when(wid == NW - 1)
    def _():
        pltpu.sync_copy(vtail_hbm, tail_v)
        pltpu.sync_copy(tail_v, pv_hbm.at[pl.ds(NVA // 4, 8)])


@functools.partial(
    pl.kernel,
    mesh=_mesh,
    out_type=jax.ShapeDtypeStruct((BATCH,), jnp.float32),
    compiler_params=pltpu.CompilerParams(needs_layout_passes=False),
    scratch_types=[
        pltpu.VMEM((BPW,), jnp.int32),
        pltpu.VMEM((BPW,), jnp.int32),
        pltpu.VMEM((BPW,), jnp.int32),
        pltpu.VMEM((BPW,), jnp.int32),
        pltpu.VMEM((2, CHUNK, PDIM), jnp.float32),
        pltpu.VMEM((2, CHUNK, PDIM), jnp.float32),
        pltpu.VMEM((BPW,), jnp.float32),
        pltpu.SemaphoreType.DMA,
        pltpu.SemaphoreType.DMA,
    ],
)
def _mf_sc(x0_hbm, x1_hbm, u_hbm, v_hbm, out_hbm,
           idx0_v, idx1_v, q0_v, q1_v, ubuf, vbuf, out_v, sem0, sem1):
    wid = lax.axis_index("s") * NC + lax.axis_index("c")
    base = wid * BPW

    pltpu.sync_copy(x0_hbm.at[pl.ds(base, BPW)], idx0_v)
    pltpu.sync_copy(x1_hbm.at[pl.ds(base, BPW)], idx1_v)

    def qbody(i, carry):
        s = pl.ds(i * L, L)
        q0_v[s] = idx0_v[s] >> 2
        q1_v[s] = idx1_v[s] >> 2
        return carry

    lax.fori_loop(0, BPW // L, qbody, 0)

    sems = (sem0, sem1)

    def fire(c):
        s = sems[c % 2]
        cp_u = pltpu.async_copy(
            u_hbm.at[q0_v.at[pl.ds(c * CHUNK, CHUNK)]], ubuf.at[c % 2], s)
        cp_v = pltpu.async_copy(
            v_hbm.at[q1_v.at[pl.ds(c * CHUNK, CHUNK)]], vbuf.at[c % 2], s)
        return cp_u, cp_v

    lane = lax.iota(jnp.int32, L)

    def compute(c):
        ub = ubuf.at[c % 2]
        vb = vbuf.at[c % 2]

        def gbody(g, carry):
            rid = g * L + lane
            s = pl.ds(c * CHUNK + g * L, L)
            off0 = (idx0_v[s] & 3) << 5
            off1 = (idx1_v[s] & 3) << 5
            acc = jnp.zeros((L,), jnp.float32)
            for d in range(DIM):
                ud = plsc.load_gather(ub, [rid, off0 + d])
                vd = plsc.load_gather(vb, [rid, off1 + d])
                acc = acc + ud * vd
            out_v[s] = acc
            return carry

        lax.fori_loop(0, GROUPS, gbody, 0)

    pending = fire(0)
    for c in range(NCHUNK):
        nxt = fire(c + 1) if c + 1 < NCHUNK else None
        pending[0].wait()
        pending[1].wait()
        compute(c)
        pending = nxt

    pltpu.sync_copy(out_v, out_hbm.at[pl.ds(base, BPW)])


def kernel(x, U, V):
    x0 = x[:, 0]
    x1 = x[:, 1]
    ut = U.T
    vt = V.T
    vtail = V[NVA:].reshape(8, PDIM)
    pu, pv = _repack(ut, vt, vtail)
    return _mf_sc(x0, x1, pu, pv)
